# baseline (speedup 1.0000x reference)
import jax
import jax.numpy as jnp
from jax.experimental import pallas as pl
from jax.experimental.pallas import tpu as pltpu

_B = 4096
_P = 1024
_Q = 1024
_E = 8
_TB = 128
_NB = _B // _TB
_K = _E * _P


def _moe_kernel(x_ref, xn_ref, wf_ref, b_ref, rw_ref, out_ref, aux_ref,
                xp0_ref, xp1_ref, wg0_ref, wg1_ref, imp_ref):
    i = pl.program_id(0)

    def router_and_build(xsrc_ref, xp_ref, wg_ref):
        x = xsrc_ref[...]
        logits = jnp.dot(x, rw_ref[...], preferred_element_type=jnp.float32)
        w = jax.nn.softmax(logits, axis=-1)
        wg_ref[...] = w
        for e in range(_E):
            xp_ref[:, e * _P:(e + 1) * _P] = x * w[:, e:e + 1]
        return jnp.sum(w, axis=0, keepdims=True)

    @pl.when(i == 0)
    def _prologue():
        imp_ref[...] = router_and_build(x_ref, xp0_ref, wg0_ref)

    def body(xp_ref, wg_ref, xpn_ref, wgn_ref):
        y = jnp.dot(xp_ref[...], wf_ref[...],
                    preferred_element_type=jnp.float32)
        out_ref[...] = y + jnp.dot(wg_ref[...], b_ref[...],
                                   preferred_element_type=jnp.float32)

        @pl.when(i < _NB - 1)
        def _build_next():
            imp_ref[...] = imp_ref[...] + router_and_build(
                xn_ref, xpn_ref, wgn_ref)

    parity = jax.lax.rem(i, 2)

    @pl.when(parity == 0)
    def _even():
        body(xp0_ref, wg0_ref, xp1_ref, wg1_ref)

    @pl.when(parity == 1)
    def _odd():
        body(xp1_ref, wg1_ref, xp0_ref, wg0_ref)

    @pl.when(i == _NB - 1)
    def _finalize():
        imp = imp_ref[...] / jnp.float32(_B)
        aux_ref[...] = jnp.float32(_E) * jnp.sum(imp * imp, keepdims=True)


def kernel(inputs, expert_w, expert_b, router_w):
    w_flat = expert_w.reshape(_K, _Q)
    out, aux = pl.pallas_call(
        _moe_kernel,
        grid=(_NB,),
        in_specs=[
            pl.BlockSpec((_TB, _P), lambda i: (i, 0)),
            pl.BlockSpec((_TB, _P), lambda i: (jnp.minimum(i + 1, _NB - 1), 0)),
            pl.BlockSpec((_K, _Q), lambda i: (0, 0)),
            pl.BlockSpec((_E, _Q), lambda i: (0, 0)),
            pl.BlockSpec((_P, _E), lambda i: (0, 0)),
        ],
        out_specs=[
            pl.BlockSpec((_TB, _Q), lambda i: (i, 0)),
            pl.BlockSpec((1, 1), lambda i: (0, 0)),
        ],
        out_shape=[
            jax.ShapeDtypeStruct((_B, _Q), jnp.float32),
            jax.ShapeDtypeStruct((1, 1), jnp.float32),
        ],
        scratch_shapes=[
            pltpu.VMEM((_TB, _K), jnp.float32),
            pltpu.VMEM((_TB, _K), jnp.float32),
            pltpu.VMEM((_TB, _E), jnp.float32),
            pltpu.VMEM((_TB, _E), jnp.float32),
            pltpu.VMEM((1, _E), jnp.float32),
        ],
        compiler_params=pltpu.CompilerParams(
            dimension_semantics=("arbitrary",),
        ),
    )(inputs, inputs, w_flat, expert_b, router_w)
    return out, aux[0, 0]
